# baseline (speedup 1.0000x reference)
import jax
import jax.numpy as jnp
from jax.experimental import pallas as pl
from jax.experimental.pallas import tpu as pltpu
from jax.experimental.pallas import tpu_sc as plsc

T, D, H, O, E, TOPK = 2048, 1024, 1024, 1024, 8, 2
BM = 256
NBLK = (T * TOPK) // BM + (E - 1)
NSLOT = NBLK * BM
TBLK = 256

_NWORK = 32
_BPW = (TOPK * T) // _NWORK
DP = D // 2


def _pack_bf16_pair(a, b):
    ua = jax.lax.bitcast_convert_type(a, jnp.uint32)
    ua = ua + jnp.uint32(0x7FFF) + ((ua >> 16) & jnp.uint32(1))
    ub = jax.lax.bitcast_convert_type(b, jnp.uint32)
    ub = ub + jnp.uint32(0x7FFF) + ((ub >> 16) & jnp.uint32(1))
    packed = (ua >> 16) | (ub & jnp.uint32(0xFFFF0000))
    return jax.lax.bitcast_convert_type(packed, jnp.int32)


def _unpack_bf16_pair(p):
    u = jax.lax.bitcast_convert_type(p, jnp.uint32)
    lo = jax.lax.bitcast_convert_type(u << 16, jnp.float32)
    hi = jax.lax.bitcast_convert_type(u & jnp.uint32(0xFFFF0000), jnp.float32)
    return lo, hi


def _gating_kernel(x_ref, G1_ref, gb1_ref, G2_ref, gb2_ref, G3_ref, gb3_ref,
                   probs_ref, loss_ref, topw_ref, dest_ref, blke_ref,
                   nused_ref, xp_ref):
    x = x_ref[...]
    xp_ref[...] = _pack_bf16_pair(x[:, :DP], x[:, DP:])
    g = jnp.dot(x, G1_ref[...], preferred_element_type=jnp.float32)
    g = jnp.maximum(g + gb1_ref[...], 0.0)
    g = jnp.dot(g, G2_ref[...], preferred_element_type=jnp.float32)
    g = jnp.maximum(g + gb2_ref[...], 0.0)
    logits = jnp.dot(g, G3_ref[...], preferred_element_type=jnp.float32)
    logits = logits + gb3_ref[...]
    m = jnp.max(logits, axis=1, keepdims=True)
    ex = jnp.exp(logits - m)
    p = ex / jnp.sum(ex, axis=1, keepdims=True)
    probs_ref[...] = p

    e_iota = jax.lax.broadcasted_iota(jnp.int32, (T, E), 1)
    m1 = jnp.max(p, axis=1, keepdims=True)
    i1 = jnp.min(jnp.where(p >= m1, e_iota, E), axis=1, keepdims=True)
    pm = jnp.where(e_iota == i1, -jnp.inf, p)
    m2 = jnp.max(pm, axis=1, keepdims=True)
    i2 = jnp.min(jnp.where(pm >= m2, e_iota, E), axis=1, keepdims=True)
    s = m1 + m2
    topw_ref[...] = jnp.concatenate([m1 / s, m2 / s], axis=1)

    oh = ((e_iota == i1) | (e_iota == i2)).astype(jnp.bfloat16)
    r0 = jax.lax.broadcasted_iota(jnp.int32, (T, T), 0)
    r1 = jax.lax.broadcasted_iota(jnp.int32, (T, T), 1)
    ltri = (r1 < r0).astype(jnp.bfloat16)
    csum = jnp.dot(ltri, oh, preferred_element_type=jnp.float32)
    counts = jnp.sum(oh.astype(jnp.float32), axis=0, keepdims=True)
    nblocks = jnp.floor((counts + (BM - 1)) * (1.0 / BM))
    padded = nblocks * BM
    c0 = jax.lax.broadcasted_iota(jnp.int32, (E, E), 0)
    c1 = jax.lax.broadcasted_iota(jnp.int32, (E, E), 1)
    estart = jnp.dot(padded, (c0 < c1).astype(jnp.float32),
                     preferred_element_type=jnp.float32)
    cumi = jnp.dot(nblocks, (c0 <= c1).astype(jnp.float32),
                   preferred_element_type=jnp.float32)
    cumi_i = cumi.astype(jnp.int32)
    nused_ref[...] = cumi_i[:, E - 1:E]
    b_iota = jax.lax.broadcasted_iota(jnp.int32, (NBLK, E), 0)
    bm = (jnp.broadcast_to(cumi_i, (NBLK, E)) <= b_iota)
    blk = jnp.sum(bm.astype(jnp.int32), axis=1, keepdims=True)
    blke_ref[...] = jnp.minimum(blk, E - 1)
    sel1 = (e_iota == i1)
    sel2 = (e_iota == i2)
    d1 = (jnp.sum(jnp.where(sel1, csum + estart, 0.0), axis=1, keepdims=True))
    d2 = (jnp.sum(jnp.where(sel2, csum + estart, 0.0), axis=1, keepdims=True))
    dest_ref[...] = jnp.concatenate([d1, d2], axis=1).astype(jnp.int32)

    usage = jnp.mean(p, axis=0, keepdims=True)
    uniform = 1.0 / E
    kl = jnp.sum(uniform * (jnp.log(uniform) - jnp.log(usage + 1e-8)))
    mu = jnp.mean(usage)
    var = jnp.sum((usage - mu) ** 2) / (E - 1)
    entropy = -jnp.sum(usage * jnp.log(usage + 1e-8))
    max_entropy = jnp.log(float(E)) * E
    total = 0.5 * kl + 0.3 * var + 0.2 * (max_entropy - entropy)
    loss_ref[...] = jnp.reshape(0.01 * total, (1, 1))


def _gating(x, G1, gb1, G2, gb2, G3, gb3):
    return pl.pallas_call(
        _gating_kernel,
        out_shape=[
            jax.ShapeDtypeStruct((T, E), jnp.float32),
            jax.ShapeDtypeStruct((1, 1), jnp.float32),
            jax.ShapeDtypeStruct((T, TOPK), jnp.float32),
            jax.ShapeDtypeStruct((T, TOPK), jnp.int32),
            jax.ShapeDtypeStruct((NBLK, 1), jnp.int32),
            jax.ShapeDtypeStruct((1, 1), jnp.int32),
            jax.ShapeDtypeStruct((T, DP), jnp.int32),
        ],
    )(x, G1, gb1.reshape(1, -1), G2, gb2.reshape(1, -1), G3,
      gb3.reshape(1, -1))


def _sc_dispatch(x, dflat):
    mesh = plsc.VectorSubcoreMesh(core_axis_name="c", subcore_axis_name="s")

    @pl.kernel(out_type=jax.ShapeDtypeStruct((NSLOT, DP), x.dtype),
               mesh=mesh,
               scratch_types=[
                   pltpu.VMEM((_BPW,), jnp.int32),
                   pltpu.VMEM((_BPW, DP), x.dtype),
                   pltpu.SemaphoreType.DMA,
               ])
    def k(x_hbm, i_hbm, o_hbm, idx_v, rows_v, sem):
        wid = jax.lax.axis_index("s") * 2 + jax.lax.axis_index("c")
        base = wid * _BPW
        src = jax.lax.rem(base, T)
        pltpu.sync_copy(i_hbm.at[pl.ds(base, _BPW)], idx_v)
        pltpu.sync_copy(x_hbm.at[pl.ds(src, _BPW)], rows_v)
        pltpu.async_copy(rows_v, o_hbm.at[idx_v], sem).wait()

    return k(x, dflat)


def _sc_collect(ys, dflat):
    OP = ys.shape[1]
    mesh = plsc.VectorSubcoreMesh(core_axis_name="c", subcore_axis_name="s")

    @pl.kernel(out_type=jax.ShapeDtypeStruct((TOPK * T, OP), ys.dtype),
               mesh=mesh,
               scratch_types=[
                   pltpu.VMEM((_BPW,), jnp.int32),
                   pltpu.VMEM((_BPW, OP), ys.dtype),
                   pltpu.SemaphoreType.DMA,
               ])
    def k(y_hbm, i_hbm, o_hbm, idx_v, rows_v, sem):
        wid = jax.lax.axis_index("s") * 2 + jax.lax.axis_index("c")
        base = wid * _BPW
        pltpu.sync_copy(i_hbm.at[pl.ds(base, _BPW)], idx_v)
        pltpu.async_copy(y_hbm.at[idx_v], rows_v, sem).wait()
        pltpu.sync_copy(rows_v, o_hbm.at[pl.ds(base, _BPW)])

    return k(ys, dflat)


def _grouped_kernel(be_ref, nu_ref, xs_ref, W1_ref, b1_ref, W2_ref, b2_ref,
                    W3_ref, b3_ref, out_ref):
    @pl.when(pl.program_id(0) < nu_ref[0])
    def _():
        xlo, xhi = _unpack_bf16_pair(xs_ref[...])
        h = (jnp.dot(xlo.astype(jnp.bfloat16), W1_ref[0, :DP],
                     preferred_element_type=jnp.float32)
             + jnp.dot(xhi.astype(jnp.bfloat16), W1_ref[0, DP:],
                       preferred_element_type=jnp.float32))
        h = jnp.maximum(h + b1_ref[0], 0.0)
        h = jnp.dot(h.astype(jnp.bfloat16), W2_ref[0],
                    preferred_element_type=jnp.float32)
        h = jnp.maximum(h + b2_ref[0], 0.0)
        y = jnp.dot(h.astype(jnp.bfloat16), W3_ref[0],
                    preferred_element_type=jnp.float32)
        y = y + b3_ref[0]
        out_ref[...] = _pack_bf16_pair(y[:, :O // 2], y[:, O // 2:])


def _grouped_mlp(blk_e, nused, xs, W1b, b1r, W2b, b2r, W3b, b3r):
    grid_spec = pltpu.PrefetchScalarGridSpec(
        num_scalar_prefetch=2,
        grid=(NBLK,),
        in_specs=[
            pl.BlockSpec((BM, DP), lambda b, be, nu: (b, 0)),
            pl.BlockSpec((1, D, H), lambda b, be, nu: (be[b], 0, 0)),
            pl.BlockSpec((1, 1, H), lambda b, be, nu: (be[b], 0, 0)),
            pl.BlockSpec((1, H, H), lambda b, be, nu: (be[b], 0, 0)),
            pl.BlockSpec((1, 1, H), lambda b, be, nu: (be[b], 0, 0)),
            pl.BlockSpec((1, H, O), lambda b, be, nu: (be[b], 0, 0)),
            pl.BlockSpec((1, 1, O), lambda b, be, nu: (be[b], 0, 0)),
        ],
        out_specs=pl.BlockSpec((BM, O // 2), lambda b, be, nu: (b, 0)),
    )
    return pl.pallas_call(
        _grouped_kernel,
        grid_spec=grid_spec,
        out_shape=jax.ShapeDtypeStruct((NSLOT, O // 2), jnp.int32),
    )(blk_e, nused, xs, W1b, b1r, W2b, b2r, W3b, b3r)


def _combine_kernel(y1_ref, y2_ref, tw_ref, out_ref):
    w1 = tw_ref[:, 0:1]
    w2 = tw_ref[:, 1:2]
    y1lo, y1hi = _unpack_bf16_pair(y1_ref[...])
    y2lo, y2hi = _unpack_bf16_pair(y2_ref[...])
    out_ref[...] = jnp.concatenate(
        [w1 * y1lo + w2 * y2lo, w1 * y1hi + w2 * y2hi], axis=1)


def _combine(ysel, topw):
    return pl.pallas_call(
        _combine_kernel,
        grid=(T // TBLK,),
        in_specs=[
            pl.BlockSpec((TBLK, O // 2), lambda i: (i, 0)),
            pl.BlockSpec((TBLK, O // 2), lambda i: (T // TBLK + i, 0)),
            pl.BlockSpec((TBLK, TOPK), lambda i: (i, 0)),
        ],
        out_specs=pl.BlockSpec((TBLK, O), lambda i: (i, 0)),
        out_shape=jax.ShapeDtypeStruct((T, O), jnp.float32),
    )(ysel, ysel, topw)


def kernel(x, W1, b1, W2, b2, W3, b3, G1, gb1, G2, gb2, G3, gb3):
    probs, loss, topw, dest, blke, nused2d, xp = _gating(
        x, G1, gb1, G2, gb2, G3, gb3)

    blk_e = blke.reshape(NBLK)
    nused = nused2d.reshape(1)
    dflat = dest.T.reshape(TOPK * T)

    xs = _sc_dispatch(xp, dflat)

    ys = _grouped_mlp(
        blk_e, nused, xs,
        W1.astype(jnp.bfloat16), b1.reshape(E, 1, H),
        W2.astype(jnp.bfloat16), b2.reshape(E, 1, H),
        W3.astype(jnp.bfloat16), b3.reshape(E, 1, O))

    ysel = _sc_collect(ys, dflat)
    out = _combine(ysel, topw)
    return out, loss.reshape(()), probs


# speedup vs baseline: 1.0072x; 1.0072x over previous
import jax
import jax.numpy as jnp
from jax.experimental import pallas as pl
from jax.experimental.pallas import tpu as pltpu
from jax.experimental.pallas import tpu_sc as plsc

T, D, H, O, E, TOPK = 2048, 1024, 1024, 1024, 8, 2
BM = 256
NBLK = (T * TOPK) // BM + (E - 1)
NSLOT = NBLK * BM
TBLK = 256

_NWORK = 32
_BPW = (TOPK * T) // _NWORK
DP = D // 2


def _pack_bf16_pair(a, b):
    ua = jax.lax.bitcast_convert_type(a, jnp.uint32)
    ua = ua + jnp.uint32(0x7FFF) + ((ua >> 16) & jnp.uint32(1))
    ub = jax.lax.bitcast_convert_type(b, jnp.uint32)
    ub = ub + jnp.uint32(0x7FFF) + ((ub >> 16) & jnp.uint32(1))
    packed = (ua >> 16) | (ub & jnp.uint32(0xFFFF0000))
    return jax.lax.bitcast_convert_type(packed, jnp.int32)


def _unpack_bf16_pair(p):
    u = jax.lax.bitcast_convert_type(p, jnp.uint32)
    lo = jax.lax.bitcast_convert_type(u << 16, jnp.float32)
    hi = jax.lax.bitcast_convert_type(u & jnp.uint32(0xFFFF0000), jnp.float32)
    return lo, hi


def _gating_kernel(x_ref, G1_ref, gb1_ref, G2_ref, gb2_ref, G3_ref, gb3_ref,
                   probs_ref, loss_ref, topw_ref, dest_ref, blke_ref,
                   nused_ref, xp_ref):
    x = x_ref[...]
    xp_ref[...] = _pack_bf16_pair(x[:, :DP], x[:, DP:])
    g = jnp.dot(x, G1_ref[...], preferred_element_type=jnp.float32)
    g = jnp.maximum(g + gb1_ref[...], 0.0)
    g = jnp.dot(g, G2_ref[...], preferred_element_type=jnp.float32)
    g = jnp.maximum(g + gb2_ref[...], 0.0)
    logits = jnp.dot(g, G3_ref[...], preferred_element_type=jnp.float32)
    logits = logits + gb3_ref[...]
    m = jnp.max(logits, axis=1, keepdims=True)
    ex = jnp.exp(logits - m)
    p = ex / jnp.sum(ex, axis=1, keepdims=True)
    probs_ref[...] = p

    e_iota = jax.lax.broadcasted_iota(jnp.int32, (T, E), 1)
    m1 = jnp.max(p, axis=1, keepdims=True)
    i1 = jnp.min(jnp.where(p >= m1, e_iota, E), axis=1, keepdims=True)
    pm = jnp.where(e_iota == i1, -jnp.inf, p)
    m2 = jnp.max(pm, axis=1, keepdims=True)
    i2 = jnp.min(jnp.where(pm >= m2, e_iota, E), axis=1, keepdims=True)
    s = m1 + m2
    topw_ref[...] = jnp.concatenate([m1 / s, m2 / s], axis=1)

    oh = ((e_iota == i1) | (e_iota == i2)).astype(jnp.bfloat16)
    CH = 256
    r0 = jax.lax.broadcasted_iota(jnp.int32, (CH, CH), 0)
    r1 = jax.lax.broadcasted_iota(jnp.int32, (CH, CH), 1)
    ltri = (r1 < r0).astype(jnp.bfloat16)
    off = jnp.zeros((1, E), jnp.float32)
    parts = []
    for c in range(T // CH):
        ohc = oh[c * CH:(c + 1) * CH]
        parts.append(jnp.dot(ltri, ohc, preferred_element_type=jnp.float32)
                     + off)
        off = off + jnp.sum(ohc.astype(jnp.float32), axis=0, keepdims=True)
    csum = jnp.concatenate(parts, axis=0)
    counts = off
    nblocks = jnp.floor((counts + (BM - 1)) * (1.0 / BM))
    padded = nblocks * BM
    c0 = jax.lax.broadcasted_iota(jnp.int32, (E, E), 0)
    c1 = jax.lax.broadcasted_iota(jnp.int32, (E, E), 1)
    estart = jnp.dot(padded, (c0 < c1).astype(jnp.float32),
                     preferred_element_type=jnp.float32)
    cumi = jnp.dot(nblocks, (c0 <= c1).astype(jnp.float32),
                   preferred_element_type=jnp.float32)
    cumi_i = cumi.astype(jnp.int32)
    nused_ref[...] = cumi_i[:, E - 1:E]
    b_iota = jax.lax.broadcasted_iota(jnp.int32, (NBLK, E), 0)
    bm = (jnp.broadcast_to(cumi_i, (NBLK, E)) <= b_iota)
    blk = jnp.sum(bm.astype(jnp.int32), axis=1, keepdims=True)
    blke_ref[...] = jnp.minimum(blk, E - 1)
    sel1 = (e_iota == i1)
    sel2 = (e_iota == i2)
    d1 = (jnp.sum(jnp.where(sel1, csum + estart, 0.0), axis=1, keepdims=True))
    d2 = (jnp.sum(jnp.where(sel2, csum + estart, 0.0), axis=1, keepdims=True))
    dest_ref[...] = jnp.concatenate([d1, d2], axis=1).astype(jnp.int32)

    usage = jnp.mean(p, axis=0, keepdims=True)
    uniform = 1.0 / E
    kl = jnp.sum(uniform * (jnp.log(uniform) - jnp.log(usage + 1e-8)))
    mu = jnp.mean(usage)
    var = jnp.sum((usage - mu) ** 2) / (E - 1)
    entropy = -jnp.sum(usage * jnp.log(usage + 1e-8))
    max_entropy = jnp.log(float(E)) * E
    total = 0.5 * kl + 0.3 * var + 0.2 * (max_entropy - entropy)
    loss_ref[...] = jnp.reshape(0.01 * total, (1, 1))


def _gating(x, G1, gb1, G2, gb2, G3, gb3):
    return pl.pallas_call(
        _gating_kernel,
        out_shape=[
            jax.ShapeDtypeStruct((T, E), jnp.float32),
            jax.ShapeDtypeStruct((1, 1), jnp.float32),
            jax.ShapeDtypeStruct((T, TOPK), jnp.float32),
            jax.ShapeDtypeStruct((T, TOPK), jnp.int32),
            jax.ShapeDtypeStruct((NBLK, 1), jnp.int32),
            jax.ShapeDtypeStruct((1, 1), jnp.int32),
            jax.ShapeDtypeStruct((T, DP), jnp.int32),
        ],
    )(x, G1, gb1.reshape(1, -1), G2, gb2.reshape(1, -1), G3,
      gb3.reshape(1, -1))


def _sc_dispatch(x, dflat):
    mesh = plsc.VectorSubcoreMesh(core_axis_name="c", subcore_axis_name="s")

    @pl.kernel(out_type=jax.ShapeDtypeStruct((NSLOT, DP), x.dtype),
               mesh=mesh,
               scratch_types=[
                   pltpu.VMEM((_BPW,), jnp.int32),
                   pltpu.VMEM((_BPW, DP), x.dtype),
                   pltpu.SemaphoreType.DMA,
               ])
    def k(x_hbm, i_hbm, o_hbm, idx_v, rows_v, sem):
        wid = jax.lax.axis_index("s") * 2 + jax.lax.axis_index("c")
        base = wid * _BPW
        src = jax.lax.rem(base, T)
        pltpu.sync_copy(i_hbm.at[pl.ds(base, _BPW)], idx_v)
        pltpu.sync_copy(x_hbm.at[pl.ds(src, _BPW)], rows_v)
        pltpu.async_copy(rows_v, o_hbm.at[idx_v], sem).wait()

    return k(x, dflat)


def _sc_collect(ys, dflat):
    OP = ys.shape[1]
    mesh = plsc.VectorSubcoreMesh(core_axis_name="c", subcore_axis_name="s")

    @pl.kernel(out_type=jax.ShapeDtypeStruct((TOPK * T, OP), ys.dtype),
               mesh=mesh,
               scratch_types=[
                   pltpu.VMEM((_BPW,), jnp.int32),
                   pltpu.VMEM((_BPW, OP), ys.dtype),
                   pltpu.SemaphoreType.DMA,
               ])
    def k(y_hbm, i_hbm, o_hbm, idx_v, rows_v, sem):
        wid = jax.lax.axis_index("s") * 2 + jax.lax.axis_index("c")
        base = wid * _BPW
        pltpu.sync_copy(i_hbm.at[pl.ds(base, _BPW)], idx_v)
        pltpu.async_copy(y_hbm.at[idx_v], rows_v, sem).wait()
        pltpu.sync_copy(rows_v, o_hbm.at[pl.ds(base, _BPW)])

    return k(ys, dflat)


def _grouped_kernel(be_ref, nu_ref, xs_ref, W1_ref, b1_ref, W2_ref, b2_ref,
                    W3_ref, b3_ref, out_ref):
    @pl.when(pl.program_id(0) < nu_ref[0])
    def _():
        xlo, xhi = _unpack_bf16_pair(xs_ref[...])
        h = (jnp.dot(xlo.astype(jnp.bfloat16), W1_ref[0, :DP],
                     preferred_element_type=jnp.float32)
             + jnp.dot(xhi.astype(jnp.bfloat16), W1_ref[0, DP:],
                       preferred_element_type=jnp.float32))
        h = jnp.maximum(h + b1_ref[0], 0.0)
        h = jnp.dot(h.astype(jnp.bfloat16), W2_ref[0],
                    preferred_element_type=jnp.float32)
        h = jnp.maximum(h + b2_ref[0], 0.0)
        y = jnp.dot(h.astype(jnp.bfloat16), W3_ref[0],
                    preferred_element_type=jnp.float32)
        y = y + b3_ref[0]
        out_ref[...] = _pack_bf16_pair(y[:, :O // 2], y[:, O // 2:])


def _grouped_mlp(blk_e, nused, xs, W1b, b1r, W2b, b2r, W3b, b3r):
    grid_spec = pltpu.PrefetchScalarGridSpec(
        num_scalar_prefetch=2,
        grid=(NBLK,),
        in_specs=[
            pl.BlockSpec((BM, DP), lambda b, be, nu: (b, 0)),
            pl.BlockSpec((1, D, H), lambda b, be, nu: (be[b], 0, 0)),
            pl.BlockSpec((1, 1, H), lambda b, be, nu: (be[b], 0, 0)),
            pl.BlockSpec((1, H, H), lambda b, be, nu: (be[b], 0, 0)),
            pl.BlockSpec((1, 1, H), lambda b, be, nu: (be[b], 0, 0)),
            pl.BlockSpec((1, H, O), lambda b, be, nu: (be[b], 0, 0)),
            pl.BlockSpec((1, 1, O), lambda b, be, nu: (be[b], 0, 0)),
        ],
        out_specs=pl.BlockSpec((BM, O // 2), lambda b, be, nu: (b, 0)),
    )
    return pl.pallas_call(
        _grouped_kernel,
        grid_spec=grid_spec,
        out_shape=jax.ShapeDtypeStruct((NSLOT, O // 2), jnp.int32),
    )(blk_e, nused, xs, W1b, b1r, W2b, b2r, W3b, b3r)


def _combine_kernel(y1_ref, y2_ref, tw_ref, out_ref):
    w1 = tw_ref[:, 0:1]
    w2 = tw_ref[:, 1:2]
    y1lo, y1hi = _unpack_bf16_pair(y1_ref[...])
    y2lo, y2hi = _unpack_bf16_pair(y2_ref[...])
    out_ref[...] = jnp.concatenate(
        [w1 * y1lo + w2 * y2lo, w1 * y1hi + w2 * y2hi], axis=1)


def _combine(ysel, topw):
    return pl.pallas_call(
        _combine_kernel,
        grid=(T // TBLK,),
        in_specs=[
            pl.BlockSpec((TBLK, O // 2), lambda i: (i, 0)),
            pl.BlockSpec((TBLK, O // 2), lambda i: (T // TBLK + i, 0)),
            pl.BlockSpec((TBLK, TOPK), lambda i: (i, 0)),
        ],
        out_specs=pl.BlockSpec((TBLK, O), lambda i: (i, 0)),
        out_shape=jax.ShapeDtypeStruct((T, O), jnp.float32),
    )(ysel, ysel, topw)


def kernel(x, W1, b1, W2, b2, W3, b3, G1, gb1, G2, gb2, G3, gb3):
    probs, loss, topw, dest, blke, nused2d, xp = _gating(
        x, G1, gb1, G2, gb2, G3, gb3)

    blk_e = blke.reshape(NBLK)
    nused = nused2d.reshape(1)
    dflat = dest.T.reshape(TOPK * T)

    xs = _sc_dispatch(xp, dflat)

    ys = _grouped_mlp(
        blk_e, nused, xs,
        W1.astype(jnp.bfloat16), b1.reshape(E, 1, H),
        W2.astype(jnp.bfloat16), b2.reshape(E, 1, H),
        W3.astype(jnp.bfloat16), b3.reshape(E, 1, O))

    ysel = _sc_collect(ys, dflat)
    out = _combine(ysel, topw)
    return out, loss.reshape(()), probs
